# baseline (speedup 1.0000x reference)
import functools

import jax
import jax.numpy as jnp
from jax import lax
from jax.experimental import pallas as pl
from jax.experimental.pallas import tpu as pltpu
from jax.experimental.pallas import tpu_sc as plsc

HIDDEN = 768
COORD = 128
MAX_2D = 1024
PAD = 1
EPS = 1e-5
CHUNK = 64
LANES = 16


def _pid_body(ids_ref, out_ref):
    ids = ids_ref[...]
    s = ids.shape[1]
    mask = (ids != PAD).astype(jnp.float32)
    iu = lax.broadcasted_iota(jnp.int32, (s, s), 0)
    it = lax.broadcasted_iota(jnp.int32, (s, s), 1)
    tri = (iu <= it).astype(jnp.float32)
    cs = jax.lax.dot(mask, tri, precision=jax.lax.Precision.HIGHEST)
    out_ref[...] = cs.astype(jnp.int32) * (ids != PAD).astype(jnp.int32) + PAD


@functools.lru_cache(maxsize=None)
def _make_pid(B, S):
    return pl.pallas_call(
        _pid_body,
        out_shape=jax.ShapeDtypeStruct((B, S), jnp.int32),
    )


def _sc_gather_builder(N, n_workers):
    tok_per_w = N // n_workers
    n_chunks = tok_per_w // CHUNK

    def body(ids_hbm, b0_hbm, b1_hbm, b2_hbm, b3_hbm,
             word_hbm, x_hbm, y_hbm, h_hbm, w_hbm,
             wout_hbm, sout_hbm,
             idc_v, b0_v, b1_v, b2_v, b3_v, hi_v, wi_v, wbuf, sbuf, sem):
        cid = lax.axis_index("c")
        sid = lax.axis_index("s")
        wid = sid * 2 + cid
        base = wid * tok_per_w

        def chunk_body(c, carry):
            tok = base + c * CHUNK
            pltpu.sync_copy(ids_hbm.at[pl.ds(tok, CHUNK)], idc_v)
            pltpu.sync_copy(b0_hbm.at[pl.ds(tok, CHUNK)], b0_v)
            pltpu.sync_copy(b1_hbm.at[pl.ds(tok, CHUNK)], b1_v)
            pltpu.sync_copy(b2_hbm.at[pl.ds(tok, CHUNK)], b2_v)
            pltpu.sync_copy(b3_hbm.at[pl.ds(tok, CHUNK)], b3_v)
            for k in range(CHUNK // LANES):
                sl = pl.ds(k * LANES, LANES)
                hi_v[sl] = jnp.clip(b3_v[sl] - b1_v[sl], 0, MAX_2D - 1)
                wi_v[sl] = jnp.clip(b2_v[sl] - b0_v[sl], 0, MAX_2D - 1)
            cps = [
                pltpu.async_copy(word_hbm.at[idc_v], wbuf, sem),
                pltpu.async_copy(x_hbm.at[b0_v], sbuf.at[:, pl.ds(0 * COORD, COORD)], sem),
                pltpu.async_copy(y_hbm.at[b1_v], sbuf.at[:, pl.ds(1 * COORD, COORD)], sem),
                pltpu.async_copy(x_hbm.at[b2_v], sbuf.at[:, pl.ds(2 * COORD, COORD)], sem),
                pltpu.async_copy(y_hbm.at[b3_v], sbuf.at[:, pl.ds(3 * COORD, COORD)], sem),
                pltpu.async_copy(h_hbm.at[hi_v], sbuf.at[:, pl.ds(4 * COORD, COORD)], sem),
                pltpu.async_copy(w_hbm.at[wi_v], sbuf.at[:, pl.ds(5 * COORD, COORD)], sem),
            ]
            for cp in cps:
                cp.wait()
            pltpu.sync_copy(wbuf, wout_hbm.at[pl.ds(tok, CHUNK)])
            pltpu.sync_copy(sbuf, sout_hbm.at[pl.ds(tok, CHUNK)])
            return carry

        lax.fori_loop(0, n_chunks, chunk_body, jnp.int32(0))

    return body


@functools.lru_cache(maxsize=None)
def _make_sc_gather(N):
    info = plsc.get_sparse_core_info()
    n_workers = info.num_cores * info.num_subcores
    mesh = plsc.VectorSubcoreMesh(core_axis_name="c", subcore_axis_name="s")
    return pl.kernel(
        _sc_gather_builder(N, n_workers),
        out_type=(jax.ShapeDtypeStruct((N, HIDDEN), jnp.float32),
                  jax.ShapeDtypeStruct((N, HIDDEN), jnp.float32)),
        mesh=mesh,
        scratch_types=[
            pltpu.VMEM((CHUNK,), jnp.int32),
            pltpu.VMEM((CHUNK,), jnp.int32),
            pltpu.VMEM((CHUNK,), jnp.int32),
            pltpu.VMEM((CHUNK,), jnp.int32),
            pltpu.VMEM((CHUNK,), jnp.int32),
            pltpu.VMEM((CHUNK,), jnp.int32),
            pltpu.VMEM((CHUNK,), jnp.int32),
            pltpu.VMEM((CHUNK, HIDDEN), jnp.float32),
            pltpu.VMEM((CHUNK, HIDDEN), jnp.float32),
            pltpu.SemaphoreType.DMA,
        ],
    )


def _ln_body(w_ref, s_ref, pid_ref, pos_ref, tte_ref, g_ref, b_ref, out_ref):
    blk = w_ref.shape[0]
    n_pos = pos_ref.shape[0]
    pid_col = jnp.swapaxes(pid_ref[0], 0, 1)
    onehot = (pid_col == lax.broadcasted_iota(jnp.int32, (blk, n_pos), 1))
    pos_rows = jax.lax.dot(onehot.astype(jnp.bfloat16), pos_ref[...],
                           preferred_element_type=jnp.float32)
    x = w_ref[...] + s_ref[...] + pos_rows + tte_ref[...]
    mu = jnp.mean(x, axis=-1, keepdims=True)
    xc = x - mu
    var = jnp.mean(xc * xc, axis=-1, keepdims=True)
    out_ref[...] = xc * lax.rsqrt(var + EPS) * g_ref[...] + b_ref[...]


@functools.lru_cache(maxsize=None)
def _make_ln(N, blk, n_pos):
    return pl.pallas_call(
        _ln_body,
        grid=(N // blk,),
        in_specs=[
            pl.BlockSpec((blk, HIDDEN), lambda i: (i, 0)),
            pl.BlockSpec((blk, HIDDEN), lambda i: (i, 0)),
            pl.BlockSpec((1, 1, blk), lambda i: (i, 0, 0)),
            pl.BlockSpec((n_pos, HIDDEN), lambda i: (0, 0)),
            pl.BlockSpec((1, HIDDEN), lambda i: (0, 0)),
            pl.BlockSpec((1, HIDDEN), lambda i: (0, 0)),
            pl.BlockSpec((1, HIDDEN), lambda i: (0, 0)),
        ],
        out_specs=pl.BlockSpec((blk, HIDDEN), lambda i: (i, 0)),
        out_shape=jax.ShapeDtypeStruct((N, HIDDEN), jnp.float32),
    )


def kernel(input_ids, bbox, word_emb, token_type_emb, pos_emb,
           x_emb, y_emb, h_emb, w_emb, ln_gamma, ln_beta):
    B, S = input_ids.shape
    N = B * S
    n_pos = pos_emb.shape[0]
    pid = _make_pid(B, S)(input_ids)
    ids = input_ids.reshape(N)
    bb = bbox.reshape(N, 4)
    wrows, srows = _make_sc_gather(N)(
        ids, bb[:, 0], bb[:, 1], bb[:, 2], bb[:, 3],
        word_emb, x_emb, y_emb, h_emb, w_emb)
    out = _make_ln(N, 512, n_pos)(
        wrows, srows, pid.reshape(N // 512, 1, 512), pos_emb.astype(jnp.bfloat16),
        token_type_emb, ln_gamma.reshape(1, HIDDEN), ln_beta.reshape(1, HIDDEN))
    return out.reshape(B, S, HIDDEN)


# speedup vs baseline: 1.0768x; 1.0768x over previous
import functools

import jax
import jax.numpy as jnp
from jax import lax
from jax.experimental import pallas as pl
from jax.experimental.pallas import tpu as pltpu
from jax.experimental.pallas import tpu_sc as plsc

HIDDEN = 768
COORD = 128
MAX_2D = 1024
PAD = 1
EPS = 1e-5
CHUNK = 32
LANES = 16


def _pid_body(ids_ref, out_ref):
    ids = ids_ref[...]
    s = ids.shape[1]
    mask = (ids != PAD).astype(jnp.float32)
    iu = lax.broadcasted_iota(jnp.int32, (s, s), 0)
    it = lax.broadcasted_iota(jnp.int32, (s, s), 1)
    tri = (iu <= it).astype(jnp.float32)
    cs = jax.lax.dot(mask, tri, precision=jax.lax.Precision.HIGHEST)
    out_ref[...] = cs.astype(jnp.int32) * (ids != PAD).astype(jnp.int32) + PAD


@functools.lru_cache(maxsize=None)
def _make_pid(B, S):
    return pl.pallas_call(
        _pid_body,
        out_shape=jax.ShapeDtypeStruct((B, S), jnp.int32),
    )


def _sc_gather_builder(N, n_workers):
    tok_per_w = N // n_workers
    n_chunks = tok_per_w // CHUNK

    def body(ids_hbm, b0_hbm, b1_hbm, b2_hbm, b3_hbm,
             word_hbm, x_hbm, y_hbm, h_hbm, w_hbm,
             wout_hbm, sout_hbm,
             ids_v, b0_v, b1_v, b2_v, b3_v, hi_v, wi_v,
             wbuf0, sbuf0, wbuf1, sbuf1, gsem0, gsem1, wsem0, wsem1):
        cid = lax.axis_index("c")
        sid = lax.axis_index("s")
        wid = sid * 2 + cid
        base = wid * tok_per_w

        pltpu.sync_copy(ids_hbm.at[pl.ds(base, tok_per_w)], ids_v)
        pltpu.sync_copy(b0_hbm.at[pl.ds(base, tok_per_w)], b0_v)
        pltpu.sync_copy(b1_hbm.at[pl.ds(base, tok_per_w)], b1_v)
        pltpu.sync_copy(b2_hbm.at[pl.ds(base, tok_per_w)], b2_v)
        pltpu.sync_copy(b3_hbm.at[pl.ds(base, tok_per_w)], b3_v)
        for k in range(tok_per_w // LANES):
            sl = pl.ds(k * LANES, LANES)
            hi_v[sl] = jnp.clip(b3_v[sl] - b1_v[sl], 0, MAX_2D - 1)
            wi_v[sl] = jnp.clip(b2_v[sl] - b0_v[sl], 0, MAX_2D - 1)

        def gather_descs(c, wbuf, sbuf, gsem):
            off = c * CHUNK
            return [
                pltpu.make_async_copy(word_hbm.at[ids_v.at[pl.ds(off, CHUNK)]], wbuf, gsem),
                pltpu.make_async_copy(x_hbm.at[b0_v.at[pl.ds(off, CHUNK)]], sbuf.at[:, pl.ds(0 * COORD, COORD)], gsem),
                pltpu.make_async_copy(y_hbm.at[b1_v.at[pl.ds(off, CHUNK)]], sbuf.at[:, pl.ds(1 * COORD, COORD)], gsem),
                pltpu.make_async_copy(x_hbm.at[b2_v.at[pl.ds(off, CHUNK)]], sbuf.at[:, pl.ds(2 * COORD, COORD)], gsem),
                pltpu.make_async_copy(y_hbm.at[b3_v.at[pl.ds(off, CHUNK)]], sbuf.at[:, pl.ds(3 * COORD, COORD)], gsem),
                pltpu.make_async_copy(h_hbm.at[hi_v.at[pl.ds(off, CHUNK)]], sbuf.at[:, pl.ds(4 * COORD, COORD)], gsem),
                pltpu.make_async_copy(w_hbm.at[wi_v.at[pl.ds(off, CHUNK)]], sbuf.at[:, pl.ds(5 * COORD, COORD)], gsem),
            ]

        def write_descs(c, wbuf, sbuf, wsem):
            tok = base + c * CHUNK
            return [
                pltpu.make_async_copy(wbuf, wout_hbm.at[pl.ds(tok, CHUNK)], wsem),
                pltpu.make_async_copy(sbuf, sout_hbm.at[pl.ds(tok, CHUNK)], wsem),
            ]

        def issue(descs):
            for d in descs:
                d.start()

        def wait(descs):
            for d in descs:
                d.wait()

        bufs = ((wbuf0, sbuf0, gsem0, wsem0), (wbuf1, sbuf1, gsem1, wsem1))

        issue(gather_descs(0, wbuf0, sbuf0, gsem0))
        issue(gather_descs(1, wbuf1, sbuf1, gsem1))
        wait(gather_descs(0, wbuf0, sbuf0, gsem0))
        issue(write_descs(0, wbuf0, sbuf0, wsem0))

        def pipe_body(i, carry):
            c0 = 2 * i
            wait(write_descs(c0 - 2, wbuf0, sbuf0, wsem0))
            issue(gather_descs(c0, wbuf0, sbuf0, gsem0))
            wait(gather_descs(c0 - 1, wbuf1, sbuf1, gsem1))
            issue(write_descs(c0 - 1, wbuf1, sbuf1, wsem1))
            wait(write_descs(c0 - 1, wbuf1, sbuf1, wsem1))
            issue(gather_descs(c0 + 1, wbuf1, sbuf1, gsem1))
            wait(gather_descs(c0, wbuf0, sbuf0, gsem0))
            issue(write_descs(c0, wbuf0, sbuf0, wsem0))
            return carry

        lax.fori_loop(1, n_chunks // 2, pipe_body, jnp.int32(0))

        wait(gather_descs(n_chunks - 1, wbuf1, sbuf1, gsem1))
        issue(write_descs(n_chunks - 1, wbuf1, sbuf1, wsem1))
        wait(write_descs(n_chunks - 2, wbuf0, sbuf0, wsem0))
        wait(write_descs(n_chunks - 1, wbuf1, sbuf1, wsem1))

    return body


@functools.lru_cache(maxsize=None)
def _make_sc_gather(N):
    info = plsc.get_sparse_core_info()
    n_workers = info.num_cores * info.num_subcores
    tok_per_w = N // n_workers
    mesh = plsc.VectorSubcoreMesh(core_axis_name="c", subcore_axis_name="s")
    return pl.kernel(
        _sc_gather_builder(N, n_workers),
        out_type=(jax.ShapeDtypeStruct((N, HIDDEN), jnp.float32),
                  jax.ShapeDtypeStruct((N, HIDDEN), jnp.float32)),
        mesh=mesh,
        scratch_types=[
            pltpu.VMEM((tok_per_w,), jnp.int32),
            pltpu.VMEM((tok_per_w,), jnp.int32),
            pltpu.VMEM((tok_per_w,), jnp.int32),
            pltpu.VMEM((tok_per_w,), jnp.int32),
            pltpu.VMEM((tok_per_w,), jnp.int32),
            pltpu.VMEM((tok_per_w,), jnp.int32),
            pltpu.VMEM((tok_per_w,), jnp.int32),
            pltpu.VMEM((CHUNK, HIDDEN), jnp.float32),
            pltpu.VMEM((CHUNK, HIDDEN), jnp.float32),
            pltpu.VMEM((CHUNK, HIDDEN), jnp.float32),
            pltpu.VMEM((CHUNK, HIDDEN), jnp.float32),
            pltpu.SemaphoreType.DMA,
            pltpu.SemaphoreType.DMA,
            pltpu.SemaphoreType.DMA,
            pltpu.SemaphoreType.DMA,
        ],
    )


def _ln_body(w_ref, s_ref, pid_ref, pos_ref, tte_ref, g_ref, b_ref, out_ref):
    blk = w_ref.shape[0]
    n_pos = pos_ref.shape[0]
    pid_col = jnp.swapaxes(pid_ref[0], 0, 1)
    onehot = (pid_col == lax.broadcasted_iota(jnp.int32, (blk, n_pos), 1))
    pos_rows = jax.lax.dot(onehot.astype(jnp.bfloat16), pos_ref[...],
                           preferred_element_type=jnp.float32)
    x = w_ref[...] + s_ref[...] + pos_rows + tte_ref[...]
    mu = jnp.mean(x, axis=-1, keepdims=True)
    xc = x - mu
    var = jnp.mean(xc * xc, axis=-1, keepdims=True)
    out_ref[...] = xc * lax.rsqrt(var + EPS) * g_ref[...] + b_ref[...]


@functools.lru_cache(maxsize=None)
def _make_ln(N, blk, n_pos):
    return pl.pallas_call(
        _ln_body,
        grid=(N // blk,),
        in_specs=[
            pl.BlockSpec((blk, HIDDEN), lambda i: (i, 0)),
            pl.BlockSpec((blk, HIDDEN), lambda i: (i, 0)),
            pl.BlockSpec((1, 1, blk), lambda i: (i, 0, 0)),
            pl.BlockSpec((n_pos, HIDDEN), lambda i: (0, 0)),
            pl.BlockSpec((1, HIDDEN), lambda i: (0, 0)),
            pl.BlockSpec((1, HIDDEN), lambda i: (0, 0)),
            pl.BlockSpec((1, HIDDEN), lambda i: (0, 0)),
        ],
        out_specs=pl.BlockSpec((blk, HIDDEN), lambda i: (i, 0)),
        out_shape=jax.ShapeDtypeStruct((N, HIDDEN), jnp.float32),
    )


def kernel(input_ids, bbox, word_emb, token_type_emb, pos_emb,
           x_emb, y_emb, h_emb, w_emb, ln_gamma, ln_beta):
    B, S = input_ids.shape
    N = B * S
    n_pos = pos_emb.shape[0]
    pid = _make_pid(B, S)(input_ids)
    ids = input_ids.reshape(N)
    bb = bbox.reshape(N, 4)
    wrows, srows = _make_sc_gather(N)(
        ids, bb[:, 0], bb[:, 1], bb[:, 2], bb[:, 3],
        word_emb, x_emb, y_emb, h_emb, w_emb)
    out = _make_ln(N, 512, n_pos)(
        wrows, srows, pid.reshape(N // 512, 1, 512), pos_emb.astype(jnp.bfloat16),
        token_type_emb, ln_gamma.reshape(1, HIDDEN), ln_beta.reshape(1, HIDDEN))
    return out.reshape(B, S, HIDDEN)
